# Initial kernel scaffold; baseline (speedup 1.0000x reference)
import jax
import jax.numpy as jnp
from jax.experimental import pallas as pl


def kernel(x, edge_attr, Wl1, bl1, Wr1, br1, We1, att1, bias1, Wl2, bl2, Wr2, br2, We2, att2, bias2, g1, be1, g2, be2, Wlin, blin, edge_index, batch):
    raise NotImplementedError("write your pallas kernel here")


# speedup vs baseline: 1.0001x; 1.0001x over previous
import jax
import jax.numpy as jnp
from jax.experimental import pallas as pl

_N = 10000
_H = 4
_C1 = 32
_C2 = 64
_G = 64


def _gatv2(x, src, dst, ea, Wl, bl, Wr, br, We, att, bias, heads, out_ch, concat):
    n = x.shape[0]
    loop = jnp.arange(n, dtype=src.dtype)
    src = jnp.concatenate([src, loop])
    dst = jnp.concatenate([dst, loop])
    loop_attr = jnp.broadcast_to(jnp.mean(ea, axis=0, keepdims=True), (n, ea.shape[1]))
    ea = jnp.concatenate([ea, loop_attr], axis=0)
    xl = (x @ Wl + bl).reshape(n, heads, out_ch)
    xr = (x @ Wr + br).reshape(n, heads, out_ch)
    xj = xl[src]
    xi = xr[dst]
    m = xi + xj + (ea @ We).reshape(-1, heads, out_ch)
    m = jax.nn.leaky_relu(m, 0.2)
    alpha = jnp.sum(m * att[None, :, :], axis=-1)
    amax = jax.ops.segment_max(alpha, dst, num_segments=n)
    alpha = jnp.exp(alpha - amax[dst])
    denom = jax.ops.segment_sum(alpha, dst, num_segments=n)
    alpha = alpha / (denom[dst] + 1e-16)
    out = jax.ops.segment_sum(xj * alpha[:, :, None], dst, num_segments=n)
    if concat:
        out = out.reshape(n, heads * out_ch)
    else:
        out = jnp.mean(out, axis=1)
    return out + bias


def _bn(x, g, b):
    mu = jnp.mean(x, axis=0)
    var = jnp.var(x, axis=0)
    return (x - mu) / jnp.sqrt(var + 1e-5) * g + b


def _final_linear_kernel(p_ref, w_ref, b_ref, o_ref):
    o_ref[...] = jnp.dot(p_ref[...], w_ref[...],
                         preferred_element_type=jnp.float32) + b_ref[...]


def kernel(x, edge_attr, Wl1, bl1, Wr1, br1, We1, att1, bias1, Wl2, bl2, Wr2, br2, We2, att2, bias2, g1, be1, g2, be2, Wlin, blin, edge_index, batch):
    src, dst = edge_index[0], edge_index[1]
    h = jax.nn.relu(_gatv2(x, src, dst, edge_attr, Wl1, bl1, Wr1, br1, We1, att1, bias1, _H, _C1, True))
    h = _bn(h, g1, be1)
    h = jax.nn.relu(_gatv2(h, src, dst, edge_attr, Wl2, bl2, Wr2, br2, We2, att2, bias2, 1, _C2, False))
    h = _bn(h, g2, be2)
    s = jax.ops.segment_sum(h, batch, num_segments=_G)
    cnt = jax.ops.segment_sum(jnp.ones((h.shape[0],), h.dtype), batch, num_segments=_G)
    mean = s / jnp.maximum(cnt, 1.0)[:, None]
    mx = jax.ops.segment_max(h, batch, num_segments=_G)
    mx = jnp.where(jnp.isfinite(mx), mx, 0.0)
    pooled = jnp.concatenate([s, mean, mx], axis=-1)
    out = pl.pallas_call(
        _final_linear_kernel,
        out_shape=jax.ShapeDtypeStruct((_G, Wlin.shape[1]), jnp.float32),
    )(pooled, Wlin, blin[None, :])
    return out + blin * 0.0


# speedup vs baseline: 18.9540x; 18.9528x over previous
import functools

import jax
import jax.numpy as jnp
import numpy as np
from jax import lax
from jax.experimental import pallas as pl
from jax.experimental.pallas import tpu as pltpu
from jax.experimental.pallas import tpu_sc as plsc

N = 10000
E = 320000
E_TOT = E + N
NW = 32
K = 64
CH = 162
EPW = CH * K
E_PAD = NW * EPW
N_PAD = 10240
ROWS_PER_TILE = N_PAD // 16
WCHUNK = 64
W = 128
DR = N_PAD // 32


def _splat_f(v):
    return lax.broadcast(jnp.float32(v), (16,))


def _splat_i(v):
    return lax.broadcast(jnp.int32(v), (16,))


def _vgather(vec, idx16):
    return lax.gather(
        vec, idx16[:, None],
        lax.GatherDimensionNumbers(offset_dims=(), collapsed_slice_dims=(0,),
                                   start_index_map=(0,)),
        (1,), mode=lax.GatherScatterMode.PROMISE_IN_BOUNDS)


def _allsum(v, lane):
    for sh in (8, 4, 2, 1):
        v = v + _vgather(v, lane ^ _splat_i(sh))
    return v


def _make_sc_layer(F, H):
    NJ = F // 16
    HJ = (F // H) // 16
    inline_den = (F + H) <= W
    roff = 0 if F == W else F
    mesh = plsc.VectorSubcoreMesh(core_axis_name="c", subcore_axis_name="s")

    out_types = [jax.ShapeDtypeStruct((2 * N_PAD, W), jnp.float32)]
    scratch = [
        pltpu.VMEM((K,), jnp.int32),
        pltpu.VMEM((K,), jnp.int32),
        pltpu.VMEM((K,), jnp.float32),
        pltpu.VMEM((F,), jnp.float32),
        pltpu.VMEM((F,), jnp.float32),
        pltpu.VMEM((K, W), jnp.float32),
        pltpu.VMEM((K, W), jnp.float32),
        pltpu.VMEM((K, W), jnp.float32),
        pltpu.VMEM_SHARED((N_PAD, W), jnp.float32),
        pltpu.SemaphoreType.DMA,
        pltpu.SemaphoreType.DMA,
    ]
    if not inline_den:
        out_types.append(jax.ShapeDtypeStruct((2 * DR, W), jnp.float32))
        scratch += [
            pltpu.VMEM((K, W), jnp.float32),
            pltpu.VMEM((K,), jnp.int32),
            pltpu.VMEM_SHARED((DR, W), jnp.float32),
        ]

    @functools.partial(
        pl.kernel,
        out_type=tuple(out_types) if len(out_types) > 1 else out_types[0],
        mesh=mesh,
        scratch_types=scratch,
    )
    def sc_layer(xl_hbm, xr_hbm, src_hbm, dst_hbm, ea_hbm, we_hbm, att_hbm,
                 *refs):
        if inline_den:
            (out_hbm, src_v, dst_v, ea_v, w_v, a_v, rows_l, rows_r, scaled,
             acc, sem1, sem2) = refs
            den_hbm = den_st = ddiv_v = dacc = None
        else:
            (out_hbm, den_hbm, src_v, dst_v, ea_v, w_v, a_v, rows_l, rows_r,
             scaled, acc, sem1, sem2, den_st, ddiv_v, dacc) = refs

        c = lax.axis_index("c")
        s = lax.axis_index("s")
        wid = c * 16 + s

        lane = lax.iota(jnp.int32, 16)
        lanef = lane.astype(jnp.float32)
        zeros16 = _splat_f(0.0)
        ones16 = _splat_f(1.0)
        c02 = _splat_f(0.2)

        def eqmask(af, bf):
            return jnp.maximum(zeros16, ones16 - jnp.abs(af - bf))

        lane0 = eqmask(lanef, zeros16)

        pltpu.sync_copy(we_hbm, w_v)
        pltpu.sync_copy(att_hbm, a_v)

        def zero_row(r, _):
            for j in range(W // 16):
                scaled[r, pl.ds(16 * j, 16)] = zeros16
            return 0

        lax.fori_loop(0, K, zero_row, 0)
        tbase = s * ROWS_PER_TILE
        for j in range(ROWS_PER_TILE // WCHUNK):
            pltpu.sync_copy(scaled.at[pl.ds(0, WCHUNK)],
                            acc.at[pl.ds(tbase + WCHUNK * j, WCHUNK)])
        if not inline_den:
            @pl.when(s < DR // 32)
            def _zero_dacc():
                pltpu.sync_copy(scaled.at[pl.ds(0, 32)],
                                dacc.at[pl.ds(s * 32, 32)])
        plsc.subcore_barrier()

        wv = [w_v[pl.ds(16 * j, 16)] for j in range(NJ)]
        av = [a_v[pl.ds(16 * j, 16)] for j in range(NJ)]

        def chunk_body(i, _):
            ebase = wid * EPW + i * K
            pltpu.sync_copy(src_hbm.at[pl.ds(ebase, K)], src_v)
            pltpu.sync_copy(dst_hbm.at[pl.ds(ebase, K)], dst_v)
            pltpu.sync_copy(ea_hbm.at[pl.ds(ebase, K)], ea_v)
            if not inline_den:
                for t in range(K // 16):
                    ddiv_v[pl.ds(16 * t, 16)] = lax.shift_right_logical(
                        dst_v[pl.ds(16 * t, 16)], _splat_i(5))
            cp1 = pltpu.async_copy(xl_hbm.at[src_v], rows_l, sem1)
            cp2 = pltpu.async_copy(xr_hbm.at[dst_v], rows_r, sem2)
            cp1.wait()
            cp2.wait()

            def edge_body(e, _):
                ebl = (e >> 4) << 4
                lidx = lax.broadcast(e, (16,)) & _splat_i(15)
                eav = _vgather(ea_v[pl.ds(ebl, 16)], lidx)
                asp = []
                for h in range(H):
                    q = None
                    for jj in range(HJ):
                        j = h * HJ + jj
                        l = rows_l[e, pl.ds(16 * j, 16)]
                        r = rows_r[e, pl.ds(roff + 16 * j, 16)]
                        m = l + r + eav * wv[j]
                        m = (jnp.maximum(m, zeros16)
                             + c02 * jnp.minimum(m, zeros16))
                        p = m * av[j]
                        q = p if q is None else q + p
                    asp.append(jnp.exp(_allsum(q, lane)))
                for j in range(NJ):
                    scaled[e, pl.ds(16 * j, 16)] = (
                        rows_l[e, pl.ds(16 * j, 16)] * asp[j // HJ])
                if inline_den:
                    scaled[e, pl.ds(F, 16)] = asp[0] * lane0
                else:
                    dsp = _vgather(dst_v[pl.ds(ebl, 16)], lidx)
                    offf = ((dsp & _splat_i(3)) * _splat_i(4)).astype(
                        jnp.float32)
                    den16 = None
                    for h in range(H):
                        t16 = asp[h] * eqmask(lanef, offf + _splat_f(h))
                        den16 = t16 if den16 is None else den16 + t16
                    jdvf = (lax.shift_right_logical(dsp, _splat_i(2))
                            & _splat_i(7)).astype(jnp.float32)
                    for jv in range(W // 16):
                        den_st[e, pl.ds(16 * jv, 16)] = den16 * eqmask(
                            jdvf, _splat_f(jv))
                return 0

            lax.fori_loop(0, K, edge_body, 0)
            pltpu.sync_copy(scaled, acc.at[dst_v], add=True)
            if not inline_den:
                pltpu.sync_copy(den_st, dacc.at[ddiv_v], add=True)
            return 0

        lax.fori_loop(0, CH, chunk_body, 0)
        plsc.subcore_barrier()

        for j in range(ROWS_PER_TILE // WCHUNK):
            r0 = tbase + WCHUNK * j
            pltpu.sync_copy(acc.at[pl.ds(r0, WCHUNK)],
                            scaled.at[pl.ds(0, WCHUNK)])
            pltpu.sync_copy(scaled.at[pl.ds(0, WCHUNK)],
                            out_hbm.at[pl.ds(c * N_PAD + r0, WCHUNK)])
        if not inline_den:
            @pl.when(s < DR // 32)
            def _write_dacc():
                pltpu.sync_copy(dacc.at[pl.ds(s * 32, 32)],
                                den_st.at[pl.ds(0, 32)])
                pltpu.sync_copy(den_st.at[pl.ds(0, 32)],
                                den_hbm.at[pl.ds(c * DR + s * 32, 32)])

    return sc_layer


_sc_layer1 = _make_sc_layer(128, 4)
_sc_layer2 = _make_sc_layer(64, 1)


def _ea_mean_body(ea_ref, o_ref):
    o_ref[...] = jnp.sum(ea_ref[...], keepdims=True) * (1.0 / E)


def _proj_body(x_ref, wl_ref, bl_ref, wr_ref, br_ref, xl_ref, xr_ref):
    x = x_ref[...]
    xl_ref[...] = jnp.dot(x, wl_ref[...], preferred_element_type=jnp.float32) + bl_ref[...]
    xr_ref[...] = jnp.dot(x, wr_ref[...], preferred_element_type=jnp.float32) + br_ref[...]


def _mid_body(feat_ref, d0_ref, d1_ref, b1_ref, g1_ref, be1_ref, wl_ref,
              bl_ref, wr_ref, br_ref, xlr_ref):
    num = feat_ref[pl.ds(0, N), :] + feat_ref[pl.ds(N_PAD, N), :]
    den4 = d0_ref[pl.ds(0, N), :] + d1_ref[pl.ds(0, N), :]
    den = jnp.concatenate(
        [jnp.broadcast_to(den4[:, h:h + 1], (N, 32)) for h in range(4)], axis=1)
    h = num / (den + 1e-16) + b1_ref[...]
    h = jnp.maximum(h, 0.0)
    mu = jnp.mean(h, axis=0, keepdims=True)
    var = jnp.mean(h * h, axis=0, keepdims=True) - mu * mu
    hn = (h - mu) / jnp.sqrt(var + 1e-5) * g1_ref[...] + be1_ref[...]
    xl = jnp.dot(hn, wl_ref[...], preferred_element_type=jnp.float32) + bl_ref[...]
    xr = jnp.dot(hn, wr_ref[...], preferred_element_type=jnp.float32) + br_ref[...]
    xlr_ref[...] = jnp.concatenate([xl, xr], axis=1)


def _final_body(acc_ref, b2_ref, g2_ref, be2_ref, batch_ref, wlin_ref, blin_ref,
                o_ref, h_ref):
    a = acc_ref[pl.ds(0, N), :]
    b = acc_ref[pl.ds(N_PAD, N), :]
    num = a[:, :64] + b[:, :64]
    den = a[:, 64:65] + b[:, 64:65]
    h = num / (den + 1e-16) + b2_ref[...]
    h = jnp.maximum(h, 0.0)
    mu = jnp.mean(h, axis=0, keepdims=True)
    var = jnp.mean(h * h, axis=0, keepdims=True) - mu * mu
    h_ref[...] = (h - mu) / jnp.sqrt(var + 1e-5) * g2_ref[...] + be2_ref[...]
    h = h_ref[...]

    batch = batch_ref[...]
    gid = lax.broadcasted_iota(jnp.int32, (1, 64), 1)
    onehot = (batch == gid).astype(jnp.float32)
    s = lax.dot_general(onehot, h, (((0,), (0,)), ((), ())),
                        preferred_element_type=jnp.float32)
    cnt = jnp.sum(onehot, axis=0)
    mean = s / jnp.maximum(cnt, 1.0)[:, None]
    big = jnp.float32(-3.4e38)
    rowsel = lax.broadcasted_iota(jnp.int32, (64, 1), 0)

    def _group_max(g, carry):
        m = jnp.max(jnp.where(batch == g, h_ref[...], big), axis=0,
                    keepdims=True)
        return jnp.where(rowsel == g, m, carry)

    mx = lax.fori_loop(0, 64, _group_max,
                       jnp.full((64, 64), big, jnp.float32))
    mx = jnp.where(cnt[:, None] > 0.0, mx, 0.0)
    pooled = jnp.concatenate([s, mean, mx], axis=-1)
    o_ref[...] = jnp.dot(pooled, wlin_ref[...],
                         preferred_element_type=jnp.float32) + blin_ref[...]


def kernel(x, edge_attr, Wl1, bl1, Wr1, br1, We1, att1, bias1, Wl2, bl2, Wr2,
           br2, We2, att2, bias2, g1, be1, g2, be2, Wlin, blin, edge_index,
           batch):
    src = edge_index[0].astype(jnp.int32)
    dst = edge_index[1].astype(jnp.int32)
    ea0 = edge_attr[:, 0]

    em = pl.pallas_call(
        _ea_mean_body,
        out_shape=jax.ShapeDtypeStruct((1, 1), jnp.float32),
    )(ea0.reshape(2500, 128))[0, 0]

    loop_idx = jnp.arange(N, dtype=jnp.int32)
    zpad_i = jnp.zeros((E_PAD - E_TOT,), jnp.int32)
    zpad_f = jnp.zeros((E_PAD - E_TOT,), jnp.float32)
    junk_dst = jnp.full((E_PAD - E_TOT,), N, jnp.int32)
    src_p = jnp.concatenate([src, loop_idx, zpad_i])
    dst_p = jnp.concatenate([dst, loop_idx, junk_dst])
    ea_p = jnp.concatenate([ea0, jnp.full((N,), em, jnp.float32), zpad_f])

    xl1, xr1 = pl.pallas_call(
        _proj_body,
        grid=(10,),
        in_specs=[
            pl.BlockSpec((1000, 128), lambda i: (i, 0)),
            pl.BlockSpec((128, 128), lambda i: (0, 0)),
            pl.BlockSpec((1, 128), lambda i: (0, 0)),
            pl.BlockSpec((128, 128), lambda i: (0, 0)),
            pl.BlockSpec((1, 128), lambda i: (0, 0)),
        ],
        out_specs=[
            pl.BlockSpec((1000, 128), lambda i: (i, 0)),
            pl.BlockSpec((1000, 128), lambda i: (i, 0)),
        ],
        out_shape=[
            jax.ShapeDtypeStruct((N, 128), jnp.float32),
            jax.ShapeDtypeStruct((N, 128), jnp.float32),
        ],
    )(x, Wl1, bl1[None, :], Wr1, br1[None, :])

    feat1, den1 = _sc_layer1(xl1, xr1, src_p, dst_p, ea_p, We1[0],
                             att1.reshape(-1))
    d0 = den1[:DR].reshape(DR * W)[:N_PAD * 4].reshape(N_PAD, 4)
    d1 = den1[DR:].reshape(DR * W)[:N_PAD * 4].reshape(N_PAD, 4)

    xlr2 = pl.pallas_call(
        _mid_body,
        out_shape=jax.ShapeDtypeStruct((N, 128), jnp.float32),
    )(feat1, d0, d1, bias1[None, :], g1[None, :], be1[None, :], Wl2,
      bl2[None, :], Wr2, br2[None, :])

    acc2 = _sc_layer2(xlr2, xlr2, src_p, dst_p, ea_p, We2[0], att2.reshape(-1))

    out = pl.pallas_call(
        _final_body,
        out_shape=jax.ShapeDtypeStruct((64, 16), jnp.float32),
        scratch_shapes=[pltpu.VMEM((N, 64), jnp.float32)],
    )(acc2, bias2[None, :], g2[None, :], be2[None, :],
      batch[:, None].astype(jnp.int32), Wlin, blin[None, :])
    return out


# speedup vs baseline: 21.7682x; 1.1485x over previous
import functools

import jax
import jax.numpy as jnp
import numpy as np
from jax import lax
from jax.experimental import pallas as pl
from jax.experimental.pallas import tpu as pltpu
from jax.experimental.pallas import tpu_sc as plsc

N = 10000
E = 320000
E_TOT = E + N
NW = 32
K = 48
CH = 216
EPW = CH * K
E_PAD = NW * EPW
N_PAD = 10240
ROWS_PER_TILE = N_PAD // 16
WCHUNK = 32
W = 128
DR = N_PAD // 32


def _splat_f(v):
    return lax.broadcast(jnp.float32(v), (16,))


def _splat_i(v):
    return lax.broadcast(jnp.int32(v), (16,))


def _vgather(vec, idx16):
    return lax.gather(
        vec, idx16[:, None],
        lax.GatherDimensionNumbers(offset_dims=(), collapsed_slice_dims=(0,),
                                   start_index_map=(0,)),
        (1,), mode=lax.GatherScatterMode.PROMISE_IN_BOUNDS)


def _allsum(v, lane):
    for sh in (8, 4, 2, 1):
        v = v + _vgather(v, lane ^ _splat_i(sh))
    return v


def _make_sc_layer(F, H):
    NJ = F // 16
    HJ = (F // H) // 16
    inline_den = (F + H) <= W
    roff = 0 if F == W else F
    mesh = plsc.VectorSubcoreMesh(core_axis_name="c", subcore_axis_name="s")

    out_types = [jax.ShapeDtypeStruct((2 * N_PAD, W), jnp.float32)]
    scratch = [
        pltpu.VMEM((F,), jnp.float32),
        pltpu.VMEM((F,), jnp.float32),
        pltpu.VMEM((K, W), jnp.float32),
        pltpu.VMEM_SHARED((N_PAD, W), jnp.float32),
    ]
    for _ in range(2):
        scratch += [
            pltpu.VMEM((K,), jnp.int32),
            pltpu.VMEM((K,), jnp.int32),
            pltpu.VMEM((K,), jnp.float32),
            pltpu.VMEM((K, W), jnp.float32),
            pltpu.VMEM((K, W), jnp.float32),
            pltpu.SemaphoreType.DMA,
        ]
    if not inline_den:
        out_types.append(jax.ShapeDtypeStruct((2 * DR, W), jnp.float32))
        scratch += [
            pltpu.VMEM((K, W), jnp.float32),
            pltpu.VMEM((K,), jnp.int32),
            pltpu.VMEM_SHARED((DR, W), jnp.float32),
        ]

    @functools.partial(
        pl.kernel,
        out_type=tuple(out_types) if len(out_types) > 1 else out_types[0],
        mesh=mesh,
        scratch_types=scratch,
    )
    def sc_layer(xl_hbm, xr_hbm, src_hbm, dst_hbm, ea_hbm, we_hbm, att_hbm,
                 *refs):
        if inline_den:
            (out_hbm, w_v, a_v, scaled, acc, *bufs) = refs
            den_hbm = den_st = ddiv_v = dacc = None
        else:
            (out_hbm, den_hbm, w_v, a_v, scaled, acc, *bufs) = refs
            den_st, ddiv_v, dacc = bufs[12:]
        sets = (tuple(bufs[0:6]), tuple(bufs[6:12]))

        c = lax.axis_index("c")
        s = lax.axis_index("s")
        wid = c * 16 + s

        lane = lax.iota(jnp.int32, 16)
        lanef = lane.astype(jnp.float32)
        zeros16 = _splat_f(0.0)
        ones16 = _splat_f(1.0)
        c02 = _splat_f(0.2)

        def eqmask(af, bf):
            return jnp.maximum(zeros16, ones16 - jnp.abs(af - bf))

        lane0 = eqmask(lanef, zeros16)

        pltpu.sync_copy(we_hbm, w_v)
        pltpu.sync_copy(att_hbm, a_v)

        def zero_row(r, _):
            for j in range(W // 16):
                scaled[r, pl.ds(16 * j, 16)] = zeros16
            return 0

        lax.fori_loop(0, K, zero_row, 0)
        tbase = s * ROWS_PER_TILE
        for j in range(ROWS_PER_TILE // WCHUNK):
            pltpu.sync_copy(scaled.at[pl.ds(0, WCHUNK)],
                            acc.at[pl.ds(tbase + WCHUNK * j, WCHUNK)])
        if not inline_den:
            @pl.when(s < DR // 32)
            def _zero_dacc():
                pltpu.sync_copy(scaled.at[pl.ds(0, 32)],
                                dacc.at[pl.ds(s * 32, 32)])
        plsc.subcore_barrier()

        wv = [w_v[pl.ds(16 * j, 16)] for j in range(NJ)]
        av = [a_v[pl.ds(16 * j, 16)] for j in range(NJ)]

        def load_and_issue(bs, ebase):
            src_v, dst_v, ea_v, rows_l, rows_r, sem = bs
            pltpu.sync_copy(src_hbm.at[pl.ds(ebase, K)], src_v)
            pltpu.sync_copy(dst_hbm.at[pl.ds(ebase, K)], dst_v)
            pltpu.sync_copy(ea_hbm.at[pl.ds(ebase, K)], ea_v)
            pltpu.async_copy(xl_hbm.at[src_v], rows_l, sem)
            pltpu.async_copy(xr_hbm.at[dst_v], rows_r, sem)

        def wait_rows(bs):
            src_v, dst_v, ea_v, rows_l, rows_r, sem = bs
            pltpu.make_async_copy(xl_hbm.at[src_v], rows_l, sem).wait()
            pltpu.make_async_copy(xr_hbm.at[dst_v], rows_r, sem).wait()

        def do_chunk(bs):
            src_v, dst_v, ea_v, rows_l, rows_r, sem = bs

            def edge_body(e, _):
                ebl = (e >> 4) << 4
                lidx = lax.broadcast(e, (16,)) & _splat_i(15)
                eav = _vgather(ea_v[pl.ds(ebl, 16)], lidx)
                asp = []
                for h in range(H):
                    q = None
                    for jj in range(HJ):
                        j = h * HJ + jj
                        l = rows_l[e, pl.ds(16 * j, 16)]
                        r = rows_r[e, pl.ds(roff + 16 * j, 16)]
                        m = l + r + eav * wv[j]
                        m = (jnp.maximum(m, zeros16)
                             + c02 * jnp.minimum(m, zeros16))
                        p = m * av[j]
                        q = p if q is None else q + p
                    asp.append(jnp.exp(_allsum(q, lane)))
                for j in range(NJ):
                    scaled[e, pl.ds(16 * j, 16)] = (
                        rows_l[e, pl.ds(16 * j, 16)] * asp[j // HJ])
                if inline_den:
                    scaled[e, pl.ds(F, 16)] = asp[0] * lane0
                else:
                    dsp = _vgather(dst_v[pl.ds(ebl, 16)], lidx)
                    offf = ((dsp & _splat_i(3)) * _splat_i(4)).astype(
                        jnp.float32)
                    den16 = None
                    for h in range(H):
                        t16 = asp[h] * eqmask(lanef, offf + _splat_f(h))
                        den16 = t16 if den16 is None else den16 + t16
                    jdvf = (lax.shift_right_logical(dsp, _splat_i(2))
                            & _splat_i(7)).astype(jnp.float32)
                    for jv in range(W // 16):
                        den_st[e, pl.ds(16 * jv, 16)] = den16 * eqmask(
                            jdvf, _splat_f(jv))
                return 0

            lax.fori_loop(0, K, edge_body, 0)
            pltpu.sync_copy(scaled, acc.at[dst_v], add=True)
            if not inline_den:
                for t in range(K // 16):
                    ddiv_v[pl.ds(16 * t, 16)] = lax.shift_right_logical(
                        dst_v[pl.ds(16 * t, 16)], _splat_i(5))
                pltpu.sync_copy(den_st, dacc.at[ddiv_v], add=True)

        ebase0 = wid * EPW
        load_and_issue(sets[0], ebase0)

        def pair_body(it, _):
            base = ebase0 + it * (2 * K)
            load_and_issue(sets[1], base + K)
            wait_rows(sets[0])
            do_chunk(sets[0])

            @pl.when(it < CH // 2 - 1)
            def _prefetch_next():
                load_and_issue(sets[0], base + 2 * K)

            wait_rows(sets[1])
            do_chunk(sets[1])
            return 0

        lax.fori_loop(0, CH // 2, pair_body, 0)
        plsc.subcore_barrier()

        for j in range(ROWS_PER_TILE // WCHUNK):
            r0 = tbase + WCHUNK * j
            pltpu.sync_copy(acc.at[pl.ds(r0, WCHUNK)],
                            scaled.at[pl.ds(0, WCHUNK)])
            pltpu.sync_copy(scaled.at[pl.ds(0, WCHUNK)],
                            out_hbm.at[pl.ds(c * N_PAD + r0, WCHUNK)])
        if not inline_den:
            @pl.when(s < DR // 32)
            def _write_dacc():
                pltpu.sync_copy(dacc.at[pl.ds(s * 32, 32)],
                                den_st.at[pl.ds(0, 32)])
                pltpu.sync_copy(den_st.at[pl.ds(0, 32)],
                                den_hbm.at[pl.ds(c * DR + s * 32, 32)])

    return sc_layer


_sc_layer1 = _make_sc_layer(128, 4)
_sc_layer2 = _make_sc_layer(64, 1)


def _ea_mean_body(ea_ref, o_ref):
    o_ref[...] = jnp.sum(ea_ref[...], keepdims=True) * (1.0 / E)


def _proj_body(x_ref, wl_ref, bl_ref, wr_ref, br_ref, xl_ref, xr_ref):
    x = x_ref[...]
    xl_ref[...] = jnp.dot(x, wl_ref[...], preferred_element_type=jnp.float32) + bl_ref[...]
    xr_ref[...] = jnp.dot(x, wr_ref[...], preferred_element_type=jnp.float32) + br_ref[...]


def _mid_body(feat_ref, d0_ref, d1_ref, b1_ref, g1_ref, be1_ref, wl_ref,
              bl_ref, wr_ref, br_ref, xlr_ref):
    num = feat_ref[pl.ds(0, N), :] + feat_ref[pl.ds(N_PAD, N), :]
    den4 = d0_ref[pl.ds(0, N), :] + d1_ref[pl.ds(0, N), :]
    den = jnp.concatenate(
        [jnp.broadcast_to(den4[:, h:h + 1], (N, 32)) for h in range(4)], axis=1)
    h = num / (den + 1e-16) + b1_ref[...]
    h = jnp.maximum(h, 0.0)
    mu = jnp.mean(h, axis=0, keepdims=True)
    var = jnp.mean(h * h, axis=0, keepdims=True) - mu * mu
    hn = (h - mu) / jnp.sqrt(var + 1e-5) * g1_ref[...] + be1_ref[...]
    xl = jnp.dot(hn, wl_ref[...], preferred_element_type=jnp.float32) + bl_ref[...]
    xr = jnp.dot(hn, wr_ref[...], preferred_element_type=jnp.float32) + br_ref[...]
    xlr_ref[...] = jnp.concatenate([xl, xr], axis=1)


def _final_body(acc_ref, b2_ref, g2_ref, be2_ref, batch_ref, wlin_ref, blin_ref,
                o_ref, h_ref):
    a = acc_ref[pl.ds(0, N), :]
    b = acc_ref[pl.ds(N_PAD, N), :]
    num = a[:, :64] + b[:, :64]
    den = a[:, 64:65] + b[:, 64:65]
    h = num / (den + 1e-16) + b2_ref[...]
    h = jnp.maximum(h, 0.0)
    mu = jnp.mean(h, axis=0, keepdims=True)
    var = jnp.mean(h * h, axis=0, keepdims=True) - mu * mu
    h_ref[...] = (h - mu) / jnp.sqrt(var + 1e-5) * g2_ref[...] + be2_ref[...]
    h = h_ref[...]

    batch = batch_ref[...]
    gid = lax.broadcasted_iota(jnp.int32, (1, 64), 1)
    onehot = (batch == gid).astype(jnp.float32)
    s = lax.dot_general(onehot, h, (((0,), (0,)), ((), ())),
                        preferred_element_type=jnp.float32)
    cnt = jnp.sum(onehot, axis=0)
    mean = s / jnp.maximum(cnt, 1.0)[:, None]
    big = jnp.float32(-3.4e38)
    rowsel = lax.broadcasted_iota(jnp.int32, (64, 1), 0)

    def _group_max(g, carry):
        m = jnp.max(jnp.where(batch == g, h_ref[...], big), axis=0,
                    keepdims=True)
        return jnp.where(rowsel == g, m, carry)

    mx = lax.fori_loop(0, 64, _group_max,
                       jnp.full((64, 64), big, jnp.float32))
    mx = jnp.where(cnt[:, None] > 0.0, mx, 0.0)
    pooled = jnp.concatenate([s, mean, mx], axis=-1)
    o_ref[...] = jnp.dot(pooled, wlin_ref[...],
                         preferred_element_type=jnp.float32) + blin_ref[...]


def kernel(x, edge_attr, Wl1, bl1, Wr1, br1, We1, att1, bias1, Wl2, bl2, Wr2,
           br2, We2, att2, bias2, g1, be1, g2, be2, Wlin, blin, edge_index,
           batch):
    src = edge_index[0].astype(jnp.int32)
    dst = edge_index[1].astype(jnp.int32)
    ea0 = edge_attr[:, 0]

    em = pl.pallas_call(
        _ea_mean_body,
        out_shape=jax.ShapeDtypeStruct((1, 1), jnp.float32),
    )(ea0.reshape(2500, 128))[0, 0]

    loop_idx = jnp.arange(N, dtype=jnp.int32)
    zpad_i = jnp.zeros((E_PAD - E_TOT,), jnp.int32)
    zpad_f = jnp.zeros((E_PAD - E_TOT,), jnp.float32)
    junk_dst = jnp.full((E_PAD - E_TOT,), N, jnp.int32)
    src_p = jnp.concatenate([src, loop_idx, zpad_i])
    dst_p = jnp.concatenate([dst, loop_idx, junk_dst])
    ea_p = jnp.concatenate([ea0, jnp.full((N,), em, jnp.float32), zpad_f])

    xl1, xr1 = pl.pallas_call(
        _proj_body,
        grid=(10,),
        in_specs=[
            pl.BlockSpec((1000, 128), lambda i: (i, 0)),
            pl.BlockSpec((128, 128), lambda i: (0, 0)),
            pl.BlockSpec((1, 128), lambda i: (0, 0)),
            pl.BlockSpec((128, 128), lambda i: (0, 0)),
            pl.BlockSpec((1, 128), lambda i: (0, 0)),
        ],
        out_specs=[
            pl.BlockSpec((1000, 128), lambda i: (i, 0)),
            pl.BlockSpec((1000, 128), lambda i: (i, 0)),
        ],
        out_shape=[
            jax.ShapeDtypeStruct((N, 128), jnp.float32),
            jax.ShapeDtypeStruct((N, 128), jnp.float32),
        ],
    )(x, Wl1, bl1[None, :], Wr1, br1[None, :])

    feat1, den1 = _sc_layer1(xl1, xr1, src_p, dst_p, ea_p, We1[0],
                             att1.reshape(-1))
    d0 = den1[:DR].reshape(DR * W)[:N_PAD * 4].reshape(N_PAD, 4)
    d1 = den1[DR:].reshape(DR * W)[:N_PAD * 4].reshape(N_PAD, 4)

    xlr2 = pl.pallas_call(
        _mid_body,
        out_shape=jax.ShapeDtypeStruct((N, 128), jnp.float32),
    )(feat1, d0, d1, bias1[None, :], g1[None, :], be1[None, :], Wl2,
      bl2[None, :], Wr2, br2[None, :])

    acc2 = _sc_layer2(xlr2, xlr2, src_p, dst_p, ea_p, We2[0], att2.reshape(-1))

    out = pl.pallas_call(
        _final_body,
        out_shape=jax.ShapeDtypeStruct((64, 16), jnp.float32),
        scratch_shapes=[pltpu.VMEM((N, 64), jnp.float32)],
    )(acc2, bias2[None, :], g2[None, :], be2[None, :],
      batch[:, None].astype(jnp.int32), Wlin, blin[None, :])
    return out


# speedup vs baseline: 22.2849x; 1.0237x over previous
import functools

import jax
import jax.numpy as jnp
import numpy as np
from jax import lax
from jax.experimental import pallas as pl
from jax.experimental.pallas import tpu as pltpu
from jax.experimental.pallas import tpu_sc as plsc

N = 10000
E = 320000
E_TOT = E + N
NW = 32
EPW = 10368
E_PAD = NW * EPW
N_PAD = 10240
ROWS_PER_TILE = N_PAD // 16
WCHUNK = 32
W = 128
DR = N_PAD // 32


def _splat_f(v):
    return lax.broadcast(jnp.float32(v), (16,))


def _splat_i(v):
    return lax.broadcast(jnp.int32(v), (16,))


def _vgather(vec, idx16):
    return lax.gather(
        vec, idx16[:, None],
        lax.GatherDimensionNumbers(offset_dims=(), collapsed_slice_dims=(0,),
                                   start_index_map=(0,)),
        (1,), mode=lax.GatherScatterMode.PROMISE_IN_BOUNDS)


def _allsum(v, lane):
    for sh in (8, 4, 2, 1):
        v = v + _vgather(v, lane ^ _splat_i(sh))
    return v


def _make_sc_layer(F, H, K, CH):
    NJ = F // 16
    HJ = (F // H) // 16
    inline_den = (F + H) <= W
    roff = 0 if F == W else F
    mesh = plsc.VectorSubcoreMesh(core_axis_name="c", subcore_axis_name="s")

    out_types = [jax.ShapeDtypeStruct((2 * N_PAD, W), jnp.float32)]
    scratch = [
        pltpu.VMEM((F,), jnp.float32),
        pltpu.VMEM((F,), jnp.float32),
        pltpu.VMEM_SHARED((N_PAD, W), jnp.float32),
    ]
    for _ in range(2):
        scratch += [
            pltpu.VMEM((K,), jnp.int32),
            pltpu.VMEM((K,), jnp.int32),
            pltpu.VMEM((K,), jnp.float32),
            pltpu.VMEM((K, W), jnp.float32),
            pltpu.VMEM((K, W), jnp.float32),
            pltpu.SemaphoreType.DMA,
        ]
    for _ in range(2):
        scratch += [
            pltpu.VMEM((K, W), jnp.float32),
            pltpu.VMEM((K,), jnp.int32),
            pltpu.SemaphoreType.DMA,
        ]
        if not inline_den:
            scratch += [
                pltpu.VMEM((K, W), jnp.float32),
                pltpu.VMEM((K,), jnp.int32),
            ]
    if not inline_den:
        out_types.append(jax.ShapeDtypeStruct((2 * DR, W), jnp.float32))
        scratch.append(pltpu.VMEM_SHARED((DR, W), jnp.float32))

    @functools.partial(
        pl.kernel,
        out_type=tuple(out_types) if len(out_types) > 1 else out_types[0],
        mesh=mesh,
        scratch_types=scratch,
    )
    def sc_layer(xl_hbm, xr_hbm, src_hbm, dst_hbm, ea_hbm, we_hbm, att_hbm,
                 *refs):
        if inline_den:
            (out_hbm, w_v, a_v, acc, *bufs) = refs
            den_hbm = dacc = None
            stagings = (tuple(bufs[12:15]), tuple(bufs[15:18]))
        else:
            (out_hbm, den_hbm, w_v, a_v, acc, *bufs) = refs
            stagings = (tuple(bufs[12:17]), tuple(bufs[17:22]))
            dacc = bufs[22]
        gsets = (tuple(bufs[0:6]), tuple(bufs[6:12]))
        scaled = stagings[0][0]

        c = lax.axis_index("c")
        s = lax.axis_index("s")
        wid = c * 16 + s

        lane = lax.iota(jnp.int32, 16)
        lanef = lane.astype(jnp.float32)
        zeros16 = _splat_f(0.0)
        ones16 = _splat_f(1.0)
        c02 = _splat_f(0.2)

        def eqmask(af, bf):
            return jnp.maximum(zeros16, ones16 - jnp.abs(af - bf))

        lane0 = eqmask(lanef, zeros16)

        pltpu.sync_copy(we_hbm, w_v)
        pltpu.sync_copy(att_hbm, a_v)

        def zero_row(r, _):
            for j in range(W // 16):
                scaled[r, pl.ds(16 * j, 16)] = zeros16
            return 0

        lax.fori_loop(0, K, zero_row, 0)
        tbase = s * ROWS_PER_TILE
        for j in range(ROWS_PER_TILE // WCHUNK):
            pltpu.sync_copy(scaled.at[pl.ds(0, WCHUNK)],
                            acc.at[pl.ds(tbase + WCHUNK * j, WCHUNK)])
        if not inline_den:
            @pl.when(s < DR // 32)
            def _zero_dacc():
                pltpu.sync_copy(scaled.at[pl.ds(0, 32)],
                                dacc.at[pl.ds(s * 32, 32)])
        plsc.subcore_barrier()

        wv = [w_v[pl.ds(16 * j, 16)] for j in range(NJ)]
        av = [a_v[pl.ds(16 * j, 16)] for j in range(NJ)]

        def load_and_issue(bs, ebase):
            src_v, dst_v, ea_v, rows_l, rows_r, sem = bs
            pltpu.sync_copy(src_hbm.at[pl.ds(ebase, K)], src_v)
            pltpu.sync_copy(dst_hbm.at[pl.ds(ebase, K)], dst_v)
            pltpu.sync_copy(ea_hbm.at[pl.ds(ebase, K)], ea_v)
            pltpu.async_copy(xl_hbm.at[src_v], rows_l, sem)
            pltpu.async_copy(xr_hbm.at[dst_v], rows_r, sem)

        def wait_rows(bs):
            src_v, dst_v, ea_v, rows_l, rows_r, sem = bs
            pltpu.make_async_copy(xl_hbm.at[src_v], rows_l, sem).wait()
            pltpu.make_async_copy(xr_hbm.at[dst_v], rows_r, sem).wait()

        def compute_chunk(bs, st):
            src_v, dst_v, ea_v, rows_l, rows_r, sem = bs
            scaled = st[0]
            den_st = st[3] if not inline_den else None

            def edge_body(e, _):
                ebl = (e >> 4) << 4
                lidx = lax.broadcast(e, (16,)) & _splat_i(15)
                eav = _vgather(ea_v[pl.ds(ebl, 16)], lidx)
                asp = []
                for h in range(H):
                    q = None
                    for jj in range(HJ):
                        j = h * HJ + jj
                        l = rows_l[e, pl.ds(16 * j, 16)]
                        r = rows_r[e, pl.ds(roff + 16 * j, 16)]
                        m = l + r + eav * wv[j]
                        m = (jnp.maximum(m, zeros16)
                             + c02 * jnp.minimum(m, zeros16))
                        p = m * av[j]
                        q = p if q is None else q + p
                    asp.append(jnp.exp(_allsum(q, lane)))
                for j in range(NJ):
                    scaled[e, pl.ds(16 * j, 16)] = (
                        rows_l[e, pl.ds(16 * j, 16)] * asp[j // HJ])
                if inline_den:
                    scaled[e, pl.ds(F, 16)] = asp[0] * lane0
                else:
                    dsp = _vgather(dst_v[pl.ds(ebl, 16)], lidx)
                    offf = ((dsp & _splat_i(3)) * _splat_i(4)).astype(
                        jnp.float32)
                    den16 = None
                    for h in range(H):
                        t16 = asp[h] * eqmask(lanef, offf + _splat_f(h))
                        den16 = t16 if den16 is None else den16 + t16
                    jdvf = (lax.shift_right_logical(dsp, _splat_i(2))
                            & _splat_i(7)).astype(jnp.float32)
                    for jv in range(W // 16):
                        den_st[e, pl.ds(16 * jv, 16)] = den16 * eqmask(
                            jdvf, _splat_f(jv))
                return 0

            lax.fori_loop(0, K, edge_body, 0)
            sdst = st[1]
            for t in range(K // 16):
                sdst[pl.ds(16 * t, 16)] = dst_v[pl.ds(16 * t, 16)]
            if not inline_den:
                ddiv_v = st[4]
                for t in range(K // 16):
                    ddiv_v[pl.ds(16 * t, 16)] = lax.shift_right_logical(
                        dst_v[pl.ds(16 * t, 16)], _splat_i(5))

        def issue_scatter(st):
            pltpu.async_copy(st[0], acc.at[st[1]], st[2], add=True)
            if not inline_den:
                pltpu.async_copy(st[3], dacc.at[st[4]], st[2], add=True)

        def wait_scatter(st):
            pltpu.make_async_copy(st[0], acc.at[st[1]], st[2]).wait()
            if not inline_den:
                pltpu.make_async_copy(st[3], dacc.at[st[4]], st[2]).wait()

        ebase0 = wid * EPW
        load_and_issue(gsets[0], ebase0)

        def pair_body(it, _):
            base = ebase0 + it * (2 * K)
            load_and_issue(gsets[1], base + K)
            wait_rows(gsets[0])

            @pl.when(it > 0)
            def _drain0():
                wait_scatter(stagings[0])

            compute_chunk(gsets[0], stagings[0])
            issue_scatter(stagings[0])

            @pl.when(it < CH // 2 - 1)
            def _prefetch_next():
                load_and_issue(gsets[0], base + 2 * K)

            wait_rows(gsets[1])

            @pl.when(it > 0)
            def _drain1():
                wait_scatter(stagings[1])

            compute_chunk(gsets[1], stagings[1])
            issue_scatter(stagings[1])
            return 0

        lax.fori_loop(0, CH // 2, pair_body, 0)
        wait_scatter(stagings[0])
        wait_scatter(stagings[1])
        plsc.subcore_barrier()

        for j in range(ROWS_PER_TILE // WCHUNK):
            r0 = tbase + WCHUNK * j
            pltpu.sync_copy(acc.at[pl.ds(r0, WCHUNK)],
                            scaled.at[pl.ds(0, WCHUNK)])
            pltpu.sync_copy(scaled.at[pl.ds(0, WCHUNK)],
                            out_hbm.at[pl.ds(c * N_PAD + r0, WCHUNK)])
        if not inline_den:
            @pl.when(s < DR // 32)
            def _write_dacc():
                pltpu.sync_copy(dacc.at[pl.ds(s * 32, 32)],
                                scaled.at[pl.ds(0, 32)])
                pltpu.sync_copy(scaled.at[pl.ds(0, 32)],
                                den_hbm.at[pl.ds(c * DR + s * 32, 32)])

    return sc_layer


_sc_layer1 = _make_sc_layer(128, 4, 32, 324)
_sc_layer2 = _make_sc_layer(64, 1, 48, 216)


def _ea_mean_body(ea_ref, o_ref):
    o_ref[...] = jnp.sum(ea_ref[...], keepdims=True) * (1.0 / E)


def _proj_body(x_ref, wl_ref, bl_ref, wr_ref, br_ref, xl_ref, xr_ref):
    x = x_ref[...]
    xl_ref[...] = jnp.dot(x, wl_ref[...], preferred_element_type=jnp.float32) + bl_ref[...]
    xr_ref[...] = jnp.dot(x, wr_ref[...], preferred_element_type=jnp.float32) + br_ref[...]


def _mid_body(feat_ref, d0_ref, d1_ref, b1_ref, g1_ref, be1_ref, wl_ref,
              bl_ref, wr_ref, br_ref, xlr_ref):
    num = feat_ref[pl.ds(0, N), :] + feat_ref[pl.ds(N_PAD, N), :]
    den4 = d0_ref[pl.ds(0, N), :] + d1_ref[pl.ds(0, N), :]
    den = jnp.concatenate(
        [jnp.broadcast_to(den4[:, h:h + 1], (N, 32)) for h in range(4)], axis=1)
    h = num / (den + 1e-16) + b1_ref[...]
    h = jnp.maximum(h, 0.0)
    mu = jnp.mean(h, axis=0, keepdims=True)
    var = jnp.mean(h * h, axis=0, keepdims=True) - mu * mu
    hn = (h - mu) / jnp.sqrt(var + 1e-5) * g1_ref[...] + be1_ref[...]
    xl = jnp.dot(hn, wl_ref[...], preferred_element_type=jnp.float32) + bl_ref[...]
    xr = jnp.dot(hn, wr_ref[...], preferred_element_type=jnp.float32) + br_ref[...]
    xlr_ref[...] = jnp.concatenate([xl, xr], axis=1)


def _final_body(acc_ref, b2_ref, g2_ref, be2_ref, batch_ref, wlin_ref, blin_ref,
                o_ref, h_ref):
    a = acc_ref[pl.ds(0, N), :]
    b = acc_ref[pl.ds(N_PAD, N), :]
    num = a[:, :64] + b[:, :64]
    den = a[:, 64:65] + b[:, 64:65]
    h = num / (den + 1e-16) + b2_ref[...]
    h = jnp.maximum(h, 0.0)
    mu = jnp.mean(h, axis=0, keepdims=True)
    var = jnp.mean(h * h, axis=0, keepdims=True) - mu * mu
    h_ref[...] = (h - mu) / jnp.sqrt(var + 1e-5) * g2_ref[...] + be2_ref[...]
    h = h_ref[...]

    batch = batch_ref[...]
    gid = lax.broadcasted_iota(jnp.int32, (1, 64), 1)
    onehot = (batch == gid).astype(jnp.float32)
    s = lax.dot_general(onehot, h, (((0,), (0,)), ((), ())),
                        preferred_element_type=jnp.float32)
    cnt = jnp.sum(onehot, axis=0)
    mean = s / jnp.maximum(cnt, 1.0)[:, None]
    big = jnp.float32(-3.4e38)
    rowsel = lax.broadcasted_iota(jnp.int32, (64, 1), 0)

    def _group_max(g, carry):
        m = jnp.max(jnp.where(batch == g, h_ref[...], big), axis=0,
                    keepdims=True)
        return jnp.where(rowsel == g, m, carry)

    mx = lax.fori_loop(0, 64, _group_max,
                       jnp.full((64, 64), big, jnp.float32))
    mx = jnp.where(cnt[:, None] > 0.0, mx, 0.0)
    pooled = jnp.concatenate([s, mean, mx], axis=-1)
    o_ref[...] = jnp.dot(pooled, wlin_ref[...],
                         preferred_element_type=jnp.float32) + blin_ref[...]


def kernel(x, edge_attr, Wl1, bl1, Wr1, br1, We1, att1, bias1, Wl2, bl2, Wr2,
           br2, We2, att2, bias2, g1, be1, g2, be2, Wlin, blin, edge_index,
           batch):
    src = edge_index[0].astype(jnp.int32)
    dst = edge_index[1].astype(jnp.int32)
    ea0 = edge_attr[:, 0]

    em = pl.pallas_call(
        _ea_mean_body,
        out_shape=jax.ShapeDtypeStruct((1, 1), jnp.float32),
    )(ea0.reshape(2500, 128))[0, 0]

    loop_idx = jnp.arange(N, dtype=jnp.int32)
    zpad_i = jnp.zeros((E_PAD - E_TOT,), jnp.int32)
    zpad_f = jnp.zeros((E_PAD - E_TOT,), jnp.float32)
    junk_dst = jnp.full((E_PAD - E_TOT,), N, jnp.int32)
    src_p = jnp.concatenate([src, loop_idx, zpad_i])
    dst_p = jnp.concatenate([dst, loop_idx, junk_dst])
    ea_p = jnp.concatenate([ea0, jnp.full((N,), em, jnp.float32), zpad_f])

    xl1, xr1 = pl.pallas_call(
        _proj_body,
        grid=(10,),
        in_specs=[
            pl.BlockSpec((1000, 128), lambda i: (i, 0)),
            pl.BlockSpec((128, 128), lambda i: (0, 0)),
            pl.BlockSpec((1, 128), lambda i: (0, 0)),
            pl.BlockSpec((128, 128), lambda i: (0, 0)),
            pl.BlockSpec((1, 128), lambda i: (0, 0)),
        ],
        out_specs=[
            pl.BlockSpec((1000, 128), lambda i: (i, 0)),
            pl.BlockSpec((1000, 128), lambda i: (i, 0)),
        ],
        out_shape=[
            jax.ShapeDtypeStruct((N, 128), jnp.float32),
            jax.ShapeDtypeStruct((N, 128), jnp.float32),
        ],
    )(x, Wl1, bl1[None, :], Wr1, br1[None, :])

    feat1, den1 = _sc_layer1(xl1, xr1, src_p, dst_p, ea_p, We1[0],
                             att1.reshape(-1))
    d0 = den1[:DR].reshape(DR * W)[:N_PAD * 4].reshape(N_PAD, 4)
    d1 = den1[DR:].reshape(DR * W)[:N_PAD * 4].reshape(N_PAD, 4)

    xlr2 = pl.pallas_call(
        _mid_body,
        out_shape=jax.ShapeDtypeStruct((N, 128), jnp.float32),
    )(feat1, d0, d1, bias1[None, :], g1[None, :], be1[None, :], Wl2,
      bl2[None, :], Wr2, br2[None, :])

    acc2 = _sc_layer2(xlr2, xlr2, src_p, dst_p, ea_p, We2[0], att2.reshape(-1))

    out = pl.pallas_call(
        _final_body,
        out_shape=jax.ShapeDtypeStruct((64, 16), jnp.float32),
        scratch_shapes=[pltpu.VMEM((N, 64), jnp.float32)],
    )(acc2, bias2[None, :], g2[None, :], be2[None, :],
      batch[:, None].astype(jnp.int32), Wlin, blin[None, :])
    return out
